# baseline (speedup 1.0000x reference)
import functools

import jax
import jax.numpy as jnp
from jax import lax
from jax.experimental import pallas as pl
from jax.experimental.pallas import tpu as pltpu

_BN_EPS = 1e-5
_VMEM_LIMIT = 48 * 1024 * 1024
_TILE_N = 16


def _silu(h):
    m = 0.5 * h
    return m + m * jnp.tanh(m)


def _lane_masks(length):
    lane = lax.broadcasted_iota(jnp.int32, (1, length), 1)
    return lane == 0, lane == length - 1


def _conv3(v, w_ref, first_mask, last_mask):
    _, ell = v.shape
    y = jnp.dot(w_ref[1], v.astype(jnp.bfloat16),
                preferred_element_type=jnp.float32)
    vm1 = jnp.where(first_mask, 0.0, pltpu.roll(v, shift=1, axis=1))
    y = y + jnp.dot(w_ref[0], vm1.astype(jnp.bfloat16),
                    preferred_element_type=jnp.float32)
    vp1 = jnp.where(last_mask, 0.0, pltpu.roll(v, shift=ell - 1, axis=1))
    y = y + jnp.dot(w_ref[2], vp1.astype(jnp.bfloat16),
                    preferred_element_type=jnp.float32)
    return y


def _fused_kernel(inv_count, has_proj, x_ref, w1_ref, b1_ref, g_ref, bt_ref,
                  w2_ref, b2_ref, *rest):
    if has_proj:
        wp_ref, o_ref, sums_ref = rest
    else:
        o_ref, sums_ref = rest
    j = pl.program_id(0)
    tn, _, ell = x_ref.shape
    first, last = _lane_masks(ell)

    @pl.when(jnp.logical_and(j == 0, pl.program_id(1) == 0))
    def _():
        sums_ref[...] = jnp.zeros_like(sums_ref)

    @pl.when(j == 0)
    def _():
        co = sums_ref.shape[0]
        acc_s = jnp.zeros((co, 1), jnp.float32)
        acc_q = jnp.zeros((co, 1), jnp.float32)
        for n in range(tn):
            h = _silu(_conv3(x_ref[n], w1_ref, first, last) + b1_ref[...])
            acc_s = acc_s + jnp.sum(h, axis=1, keepdims=True)
            acc_q = acc_q + jnp.sum(h * h, axis=1, keepdims=True)
        sums_ref[:, 0:1] += acc_s
        sums_ref[:, 1:2] += acc_q

    @pl.when(j == 1)
    def _():
        mean = sums_ref[:, 0:1] * inv_count
        var = sums_ref[:, 1:2] * inv_count - mean * mean
        scale = g_ref[...] * lax.rsqrt(var + _BN_EPS)
        shift = bt_ref[...] - mean * scale
        for n in range(tn):
            x_n = x_ref[n]
            if has_proj:
                p = jnp.dot(wp_ref[...], x_n.astype(jnp.bfloat16),
                            preferred_element_type=jnp.float32)
            h = _silu(_conv3(x_n, w1_ref, first, last) + b1_ref[...])
            h = h * scale + shift
            y = _conv3(h, w2_ref, first, last) + b2_ref[...]
            if has_proj:
                y = y + p
            else:
                y = y + x_n
            o_ref[n] = y


def _conv_weight(w):
    return jnp.transpose(w, (2, 0, 1)).astype(jnp.bfloat16)


def _const_spec(shape):
    rank = len(shape)
    return pl.BlockSpec(shape, lambda *_, _r=rank: (0,) * _r)


def _tile(n, want):
    tn = min(n, want)
    while n % tn:
        tn -= 1
    return tn


def kernel(x, w1, b1, gamma, beta, w2, b2, wp, bp):
    n, ci, ell = x.shape
    co = w1.shape[0]
    has_proj = wp is not None

    w1_k = _conv_weight(w1)
    w2_k = _conv_weight(w2)
    b1_2 = b1.reshape(co, 1)
    b2_2 = (b2 + (bp if has_proj else 0.0)).reshape(co, 1)
    g_2 = gamma.reshape(co, 1)
    bt_2 = beta.reshape(co, 1)
    inv_count = 1.0 / float(n * ell)

    tn = _tile(n, _TILE_N)
    g = n // tn
    x_spec = pl.BlockSpec((tn, ci, ell), lambda j, i: (i, 0, 0))
    ins = [x, w1_k, b1_2, g_2, bt_2, w2_k, b2_2]
    in_specs = [x_spec, _const_spec(w1_k.shape), _const_spec(b1_2.shape),
                _const_spec(g_2.shape), _const_spec(bt_2.shape),
                _const_spec(w2_k.shape), _const_spec(b2_2.shape)]
    if has_proj:
        wp_2 = wp[:, :, 0].astype(jnp.bfloat16)
        ins.append(wp_2)
        in_specs.append(_const_spec(wp_2.shape))

    return pl.pallas_call(
        functools.partial(_fused_kernel, inv_count, has_proj),
        out_shape=jax.ShapeDtypeStruct((n, co, ell), jnp.float32),
        grid=(2, g),
        in_specs=in_specs,
        out_specs=pl.BlockSpec((tn, co, ell),
                               lambda j, i: (jnp.where(j == 0, 0, i), 0, 0)),
        scratch_shapes=[pltpu.VMEM((co, 128), jnp.float32)],
        compiler_params=pltpu.CompilerParams(
            dimension_semantics=("arbitrary", "arbitrary"),
            vmem_limit_bytes=_VMEM_LIMIT),
    )(*ins)


# speedup vs baseline: 1.0839x; 1.0839x over previous
import functools

import jax
import jax.numpy as jnp
from jax import lax
from jax.experimental import pallas as pl
from jax.experimental.pallas import tpu as pltpu

_BN_EPS = 1e-5
_VMEM_LIMIT = 48 * 1024 * 1024
_TILE_STATS = 32
_TILE_APPLY = 16


def _silu_bf16(c, b_ref):
    t = c.astype(jnp.bfloat16) + b_ref[...]
    m = jnp.bfloat16(0.5) * t
    return m + m * jnp.tanh(m)


def _lane_masks(length):
    lane = lax.broadcasted_iota(jnp.int32, (1, length), 1)
    return lane == 0, lane == length - 1


def _conv3(vb, w_ref, first_mask, last_mask):
    _, ell = vb.shape
    y = jnp.dot(w_ref[1], vb, preferred_element_type=jnp.float32)
    vm1 = jnp.where(first_mask, 0, pltpu.roll(vb, shift=1, axis=1))
    y = y + jnp.dot(w_ref[0], vm1, preferred_element_type=jnp.float32)
    vp1 = jnp.where(last_mask, 0, pltpu.roll(vb, shift=ell - 1, axis=1))
    y = y + jnp.dot(w_ref[2], vp1, preferred_element_type=jnp.float32)
    return y


def _stats_kernel(x_ref, w1_ref, b1_ref, s_ref):
    tn, _, ell = x_ref.shape
    co = s_ref.shape[1]
    first, last = _lane_masks(ell)
    acc_s = jnp.zeros((co, 1), jnp.float32)
    acc_q = jnp.zeros((co, 1), jnp.float32)
    for n in range(tn):
        xb = x_ref[n].astype(jnp.bfloat16)
        h = _silu_bf16(_conv3(xb, w1_ref, first, last), b1_ref).astype(jnp.float32)
        acc_s = acc_s + jnp.sum(h, axis=1, keepdims=True)
        acc_q = acc_q + jnp.sum(h * h, axis=1, keepdims=True)
    s_ref[0, :, 0:1] = acc_s
    s_ref[0, :, 1:2] = acc_q


def _apply_kernel(has_proj, x_ref, w1_ref, b1_ref, w2_ref, plane_ref, *rest):
    if has_proj:
        wp_ref, o_ref = rest
    else:
        (o_ref,) = rest
    tn, _, ell = x_ref.shape
    first, last = _lane_masks(ell)
    for n in range(tn):
        xb = x_ref[n].astype(jnp.bfloat16)
        if has_proj:
            p = jnp.dot(wp_ref[...], xb, preferred_element_type=jnp.float32)
        hb = _silu_bf16(_conv3(xb, w1_ref, first, last), b1_ref)
        y = _conv3(hb, w2_ref, first, last) + plane_ref[...]
        if has_proj:
            y = y + p
        else:
            y = y + x_ref[n]
        o_ref[n] = y


def _const_spec(shape):
    rank = len(shape)
    return pl.BlockSpec(shape, lambda *_, _r=rank: (0,) * _r)


def _tile(n, want):
    tn = min(n, want)
    while n % tn:
        tn -= 1
    return tn


def kernel(x, w1, b1, gamma, beta, w2, b2, wp, bp):
    n, ci, ell = x.shape
    co = w1.shape[0]
    has_proj = wp is not None

    w1_k = jnp.transpose(w1, (2, 0, 1)).astype(jnp.bfloat16)
    w2_f = jnp.transpose(w2, (2, 0, 1))
    b1_b = b1.reshape(co, 1).astype(jnp.bfloat16)

    tn1 = _tile(n, _TILE_STATS)
    g1 = n // tn1
    stats = pl.pallas_call(
        _stats_kernel,
        out_shape=jax.ShapeDtypeStruct((g1, co, 128), jnp.float32),
        grid=(g1,),
        in_specs=[pl.BlockSpec((tn1, ci, ell), lambda i: (i, 0, 0)),
                  _const_spec(w1_k.shape), _const_spec(b1_b.shape)],
        out_specs=pl.BlockSpec((1, co, 128), lambda i: (i, 0, 0)),
        compiler_params=pltpu.CompilerParams(
            dimension_semantics=("arbitrary",), vmem_limit_bytes=_VMEM_LIMIT),
    )(x, w1_k, b1_b)

    inv_count = 1.0 / float(n * ell)
    mean = jnp.sum(stats[:, :, 0], axis=0).reshape(co, 1) * inv_count
    var = jnp.sum(stats[:, :, 1], axis=0).reshape(co, 1) * inv_count - mean * mean
    scale = gamma.reshape(co, 1) * lax.rsqrt(var + _BN_EPS)
    shift = beta.reshape(co, 1) - mean * scale

    w2_s = (w2_f * scale.reshape(1, 1, co)).astype(jnp.bfloat16)
    t_taps = jnp.einsum("koc,cx->kox", w2_f, shift)
    bias_col = t_taps.sum(0) + b2.reshape(co, 1)
    if has_proj:
        bias_col = bias_col + bp.reshape(co, 1)
    lane = lax.broadcasted_iota(jnp.int32, (1, ell), 1)
    plane = (bias_col
             - jnp.where(lane == 0, 1.0, 0.0) * t_taps[0]
             - jnp.where(lane == ell - 1, 1.0, 0.0) * t_taps[2])

    tn2 = _tile(n, _TILE_APPLY)
    g2 = n // tn2
    x_spec = pl.BlockSpec((tn2, ci, ell), lambda i: (i, 0, 0))
    ins = [x, w1_k, b1_b, w2_s, plane]
    in_specs = [x_spec, _const_spec(w1_k.shape), _const_spec(b1_b.shape),
                _const_spec(w2_s.shape), _const_spec(plane.shape)]
    if has_proj:
        wp_2 = wp[:, :, 0].astype(jnp.bfloat16)
        ins.append(wp_2)
        in_specs.append(_const_spec(wp_2.shape))

    return pl.pallas_call(
        functools.partial(_apply_kernel, has_proj),
        out_shape=jax.ShapeDtypeStruct((n, co, ell), jnp.float32),
        grid=(g2,),
        in_specs=in_specs,
        out_specs=pl.BlockSpec((tn2, co, ell), lambda i: (i, 0, 0)),
        compiler_params=pltpu.CompilerParams(
            dimension_semantics=("arbitrary",), vmem_limit_bytes=_VMEM_LIMIT),
    )(*ins)


# speedup vs baseline: 1.1310x; 1.0435x over previous
import functools

import jax
import jax.numpy as jnp
from jax import lax
from jax.experimental import pallas as pl
from jax.experimental.pallas import tpu as pltpu

_BN_EPS = 1e-5
_VMEM_LIMIT = 48 * 1024 * 1024
_TILE_STATS = 32
_TILE_APPLY = 16


def _silu_bf16(c, b_ref):
    t = c.astype(jnp.bfloat16) + b_ref[...]
    m = jnp.bfloat16(0.5) * t
    return m + m * jnp.tanh(m)


def _lane_masks(length):
    lane = lax.broadcasted_iota(jnp.int32, (1, length), 1)
    return lane == 0, lane == length - 1


def _taps(vb, first_mask, last_mask):
    ell = vb.shape[-1]
    vm1 = jnp.where(first_mask, 0, pltpu.roll(vb, shift=1, axis=1))
    vp1 = jnp.where(last_mask, 0, pltpu.roll(vb, shift=ell - 1, axis=1))
    return vm1, vp1


def _stats_kernel(x_ref, w1_ref, b1_ref, s_ref):
    tn, _, ell = x_ref.shape
    co = s_ref.shape[1]
    ci = x_ref.shape[1]
    first, last = _lane_masks(ell)
    acc_s = jnp.zeros((co, 1), jnp.float32)
    acc_q = jnp.zeros((co, 1), jnp.float32)
    for n in range(tn):
        xb = x_ref[n].astype(jnp.bfloat16)
        vm1, vp1 = _taps(xb, first, last)
        c1 = (jnp.dot(w1_ref[:, ci:2 * ci], xb,
                      preferred_element_type=jnp.float32)
              + jnp.dot(w1_ref[:, :ci], vm1,
                        preferred_element_type=jnp.float32)
              + jnp.dot(w1_ref[:, 2 * ci:], vp1,
                        preferred_element_type=jnp.float32))
        h = _silu_bf16(c1, b1_ref).astype(jnp.float32)
        acc_s = acc_s + jnp.sum(h, axis=1, keepdims=True)
        acc_q = acc_q + jnp.sum(h * h, axis=1, keepdims=True)
    s_ref[0, :, 0:1] = acc_s
    s_ref[0, :, 1:2] = acc_q


def _apply_kernel(has_proj, x_ref, w1_ref, b1_ref, w2_ref, plane_ref, o_ref):
    tn, _, ell = x_ref.shape
    first, last = _lane_masks(ell)
    for n in range(tn):
        xb = x_ref[n].astype(jnp.bfloat16)
        vm1, vp1 = _taps(xb, first, last)
        c1 = jnp.dot(w1_ref[...], jnp.concatenate([vm1, xb, vp1], axis=0),
                     preferred_element_type=jnp.float32)
        hb = _silu_bf16(c1, b1_ref)
        hm, hp = _taps(hb, first, last)
        parts = [hm, hb, hp] + ([xb] if has_proj else [])
        y = jnp.dot(w2_ref[...], jnp.concatenate(parts, axis=0),
                    preferred_element_type=jnp.float32) + plane_ref[...]
        if not has_proj:
            y = y + x_ref[n]
        o_ref[n] = y


def _const_spec(shape):
    rank = len(shape)
    return pl.BlockSpec(shape, lambda *_, _r=rank: (0,) * _r)


def _tile(n, want):
    tn = min(n, want)
    while n % tn:
        tn -= 1
    return tn


def kernel(x, w1, b1, gamma, beta, w2, b2, wp, bp):
    n, ci, ell = x.shape
    co = w1.shape[0]
    has_proj = wp is not None

    w1_k = jnp.transpose(w1, (0, 2, 1)).reshape(co, 3 * ci).astype(jnp.bfloat16)
    w2_f = jnp.transpose(w2, (2, 0, 1))
    b1_b = b1.reshape(co, 1).astype(jnp.bfloat16)

    tn1 = _tile(n, _TILE_STATS)
    g1 = n // tn1
    stats = pl.pallas_call(
        _stats_kernel,
        out_shape=jax.ShapeDtypeStruct((g1, co, 128), jnp.float32),
        grid=(g1,),
        in_specs=[pl.BlockSpec((tn1, ci, ell), lambda i: (i, 0, 0)),
                  _const_spec(w1_k.shape), _const_spec(b1_b.shape)],
        out_specs=pl.BlockSpec((1, co, 128), lambda i: (i, 0, 0)),
        compiler_params=pltpu.CompilerParams(
            dimension_semantics=("arbitrary",), vmem_limit_bytes=_VMEM_LIMIT),
    )(x, w1_k, b1_b)

    inv_count = 1.0 / float(n * ell)
    mean = jnp.sum(stats[:, :, 0], axis=0).reshape(co, 1) * inv_count
    var = jnp.sum(stats[:, :, 1], axis=0).reshape(co, 1) * inv_count - mean * mean
    scale = gamma.reshape(co, 1) * lax.rsqrt(var + _BN_EPS)
    shift = beta.reshape(co, 1) - mean * scale

    w2_s = jnp.transpose(w2_f * scale.reshape(1, 1, co),
                         (1, 0, 2)).reshape(co, 3 * co)
    if has_proj:
        w2_s = jnp.concatenate([w2_s, wp[:, :, 0]], axis=1)
    w2_s = w2_s.astype(jnp.bfloat16)
    t_taps = jnp.einsum("koc,cx->kox", w2_f, shift)
    bias_col = t_taps.sum(0) + b2.reshape(co, 1)
    if has_proj:
        bias_col = bias_col + bp.reshape(co, 1)
    lane = lax.broadcasted_iota(jnp.int32, (1, ell), 1)
    plane = (bias_col
             - jnp.where(lane == 0, 1.0, 0.0) * t_taps[0]
             - jnp.where(lane == ell - 1, 1.0, 0.0) * t_taps[2])

    tn2 = _tile(n, _TILE_APPLY)
    g2 = n // tn2
    x_spec = pl.BlockSpec((tn2, ci, ell), lambda i: (i, 0, 0))
    ins = [x, w1_k, b1_b, w2_s, plane]
    in_specs = [x_spec, _const_spec(w1_k.shape), _const_spec(b1_b.shape),
                _const_spec(w2_s.shape), _const_spec(plane.shape)]

    return pl.pallas_call(
        functools.partial(_apply_kernel, has_proj),
        out_shape=jax.ShapeDtypeStruct((n, co, ell), jnp.float32),
        grid=(g2,),
        in_specs=in_specs,
        out_specs=pl.BlockSpec((tn2, co, ell), lambda i: (i, 0, 0)),
        compiler_params=pltpu.CompilerParams(
            dimension_semantics=("arbitrary",), vmem_limit_bytes=_VMEM_LIMIT),
    )(*ins)
